# baseline (speedup 1.0000x reference)
import functools

import jax
import jax.numpy as jnp
from jax import lax
from jax.experimental import pallas as pl
from jax.experimental.pallas import tpu as pltpu
from jax.experimental.pallas import tpu_sc as plsc

N = 10000
E = 320000
D = 128
H = 256
C = 47
CP = 48

NPAD = 10240
NTILE = 32
EPAD = 327680
EPT = EPAD // NTILE
BB = 128
KB = EPT // BB
EPS = EPAD // 16
KBC = EPS // BB
NBUF = 4
RPS = NPAD // 16
DH = D // 2

_mesh = plsc.VectorSubcoreMesh(core_axis_name="c", subcore_axis_name="s")


def _zero_vmem_2d(zbuf, rows, cols):
    z16 = jnp.zeros((16,), jnp.float32)
    for r in range(rows):
        for c in range(cols // 16):
            zbuf[r, pl.ds(c * 16, 16)] = z16


@functools.partial(
    pl.kernel,
    mesh=_mesh,
    out_type=(
        jax.ShapeDtypeStruct((NPAD,), jnp.float32),
        jax.ShapeDtypeStruct((NPAD,), jnp.float32),
    ),
    scratch_types=[
        pltpu.VMEM((KB, BB), jnp.int32),
        pltpu.VMEM((BB,), jnp.float32),
        pltpu.VMEM((RPS,), jnp.float32),
        pltpu.VMEM_SHARED((NPAD,), jnp.float32),
        pltpu.SemaphoreType.DMA,
    ],
)
def _deg_kernel(dst_hbm, out0, out1, didx, ones_v, zrow, acc, sem):
    cid = lax.axis_index("c")
    sid = lax.axis_index("s")
    wid = cid * 16 + sid
    for i in range(BB // 16):
        ones_v[pl.ds(i * 16, 16)] = jnp.ones((16,), jnp.float32)
    for i in range(RPS // 16):
        zrow[pl.ds(i * 16, 16)] = jnp.zeros((16,), jnp.float32)
    pltpu.sync_copy(dst_hbm.at[wid], didx)
    pltpu.sync_copy(zrow, acc.at[pl.ds(sid * RPS, RPS)])
    plsc.subcore_barrier()

    @pl.loop(0, KB)
    def _(j):
        pltpu.async_copy(ones_v, acc.at[didx.at[j]], sem, add=True)

    @pl.loop(0, KB)
    def _(j):
        pltpu.make_async_copy(ones_v, acc.at[didx.at[j]], sem).wait()

    plsc.subcore_barrier()

    @pl.when(cid == 0)
    def _():
        pltpu.sync_copy(acc.at[pl.ds(sid * RPS, RPS)],
                        out0.at[pl.ds(sid * RPS, RPS)])

    @pl.when(cid == 1)
    def _():
        pltpu.sync_copy(acc.at[pl.ds(sid * RPS, RPS)],
                        out1.at[pl.ds(sid * RPS, RPS)])


def _make_agg_kernel(width):
    @functools.partial(
        pl.kernel,
        mesh=_mesh,
        out_type=(
            jax.ShapeDtypeStruct((NPAD, width), jnp.float32),
            jax.ShapeDtypeStruct((NPAD, width), jnp.float32),
        ),
        scratch_types=[
            pltpu.VMEM((KB, BB), jnp.int32),
            pltpu.VMEM((KB, BB), jnp.int32),
            [pltpu.VMEM((BB, width), jnp.float32)] * NBUF,
            pltpu.VMEM((16, width), jnp.float32),
            pltpu.VMEM_SHARED((NPAD, width), jnp.float32),
            [pltpu.SemaphoreType.DMA] * NBUF,
        ],
        compiler_params=pltpu.CompilerParams(use_tc_tiling_on_sc=False),
    )
    def agg(src_hbm, dst_hbm, x_hbm, out0, out1, sidx, didx, rows, zbuf,
            acc, gsem):
        cid = lax.axis_index("c")
        sid = lax.axis_index("s")
        wid = cid * 16 + sid
        _zero_vmem_2d(zbuf, 16, width)
        for t in range(RPS // 16):
            pltpu.sync_copy(zbuf, acc.at[pl.ds(sid * RPS + t * 16, 16)])
        pltpu.sync_copy(src_hbm.at[wid], sidx)
        pltpu.sync_copy(dst_hbm.at[wid], didx)
        plsc.subcore_barrier()

        for b in range(NBUF):
            pltpu.async_copy(x_hbm.at[sidx.at[b]], rows[b], gsem[b])

        @pl.loop(0, KB // NBUF - 1)
        def _(g):
            j0 = g * NBUF
            for b in range(NBUF):
                pltpu.make_async_copy(
                    x_hbm.at[sidx.at[j0 + b]], rows[b], gsem[b]).wait()
                pltpu.sync_copy(rows[b], acc.at[didx.at[j0 + b]], add=True)
                pltpu.async_copy(
                    x_hbm.at[sidx.at[j0 + NBUF + b]], rows[b], gsem[b])

        for b in range(NBUF):
            j = KB - NBUF + b
            pltpu.make_async_copy(
                x_hbm.at[sidx.at[j]], rows[b], gsem[b]).wait()
            pltpu.sync_copy(rows[b], acc.at[didx.at[j]], add=True)

        plsc.subcore_barrier()

        @pl.when(cid == 0)
        def _():
            pltpu.sync_copy(acc.at[pl.ds(sid * RPS, RPS)],
                            out0.at[pl.ds(sid * RPS, RPS)])

        @pl.when(cid == 1)
        def _():
            pltpu.sync_copy(acc.at[pl.ds(sid * RPS, RPS)],
                            out1.at[pl.ds(sid * RPS, RPS)])

    return agg


_agg_c = _make_agg_kernel(CP)


@functools.partial(
    pl.kernel,
    mesh=_mesh,
    out_type=(
        jax.ShapeDtypeStruct((NPAD, DH), jnp.float32),
        jax.ShapeDtypeStruct((NPAD, DH), jnp.float32),
    ),
    scratch_types=[
        pltpu.VMEM((KBC, BB), jnp.int32),
        pltpu.VMEM((KBC, BB), jnp.int32),
        [pltpu.VMEM((BB, DH), jnp.float32)] * NBUF,
        pltpu.VMEM((16, DH), jnp.float32),
        pltpu.VMEM_SHARED((NPAD, DH), jnp.float32),
        [pltpu.SemaphoreType.DMA] * NBUF,
    ],
    compiler_params=pltpu.CompilerParams(use_tc_tiling_on_sc=False),
)
def _agg_d(src_hbm, dst_hbm, x0_hbm, x1_hbm, out0, out1, sidx, didx, rows,
           zbuf, acc, gsem):
    cid = lax.axis_index("c")
    sid = lax.axis_index("s")
    _zero_vmem_2d(zbuf, 16, DH)
    for t in range(RPS // 16):
        pltpu.sync_copy(zbuf, acc.at[pl.ds(sid * RPS + t * 16, 16)])
    pltpu.sync_copy(src_hbm.at[sid], sidx)
    pltpu.sync_copy(dst_hbm.at[sid], didx)
    plsc.subcore_barrier()

    def run(x_hbm):
        for b in range(NBUF):
            pltpu.async_copy(x_hbm.at[sidx.at[b]], rows[b], gsem[b])

        @pl.loop(0, KBC // NBUF - 1)
        def _(g):
            j0 = g * NBUF
            for b in range(NBUF):
                pltpu.make_async_copy(
                    x_hbm.at[sidx.at[j0 + b]], rows[b], gsem[b]).wait()
                pltpu.sync_copy(rows[b], acc.at[didx.at[j0 + b]], add=True)
                pltpu.async_copy(
                    x_hbm.at[sidx.at[j0 + NBUF + b]], rows[b], gsem[b])

        for b in range(NBUF):
            j = KBC - NBUF + b
            pltpu.make_async_copy(
                x_hbm.at[sidx.at[j]], rows[b], gsem[b]).wait()
            pltpu.sync_copy(rows[b], acc.at[didx.at[j]], add=True)

    @pl.when(cid == 0)
    def _():
        run(x0_hbm)

    @pl.when(cid == 1)
    def _():
        run(x1_hbm)

    plsc.subcore_barrier()

    @pl.when(cid == 0)
    def _():
        pltpu.sync_copy(acc.at[pl.ds(sid * RPS, RPS)],
                        out0.at[pl.ds(sid * RPS, RPS)])

    @pl.when(cid == 1)
    def _():
        pltpu.sync_copy(acc.at[pl.ds(sid * RPS, RPS)],
                        out1.at[pl.ds(sid * RPS, RPS)])


_R = 512
_GRID = NPAD // _R


def _scale_in_body(x_ref, d0_ref, d1_ref, xt0_ref, xt1_ref, rdeg_ref):
    deg = jnp.maximum(d0_ref[...] + d1_ref[...], 1.0)
    rd = lax.rsqrt(deg)
    rdeg_ref[...] = rd
    xt = x_ref[...] * rd
    xt0_ref[...] = xt[:, :DH]
    xt1_ref[...] = xt[:, DH:]


def _scale_in(x_pad, deg0, deg1):
    return pl.pallas_call(
        _scale_in_body,
        grid=(_GRID,),
        in_specs=[
            pl.BlockSpec((_R, D), lambda i: (i, 0)),
            pl.BlockSpec((_R, 1), lambda i: (i, 0)),
            pl.BlockSpec((_R, 1), lambda i: (i, 0)),
        ],
        out_specs=[
            pl.BlockSpec((_R, DH), lambda i: (i, 0)),
            pl.BlockSpec((_R, DH), lambda i: (i, 0)),
            pl.BlockSpec((_R, 1), lambda i: (i, 0)),
        ],
        out_shape=[
            jax.ShapeDtypeStruct((NPAD, DH), jnp.float32),
            jax.ShapeDtypeStruct((NPAD, DH), jnp.float32),
            jax.ShapeDtypeStruct((NPAD, 1), jnp.float32),
        ],
    )(x_pad, deg0, deg1)


def _mid_body(a0_ref, a1_ref, rd_ref, w1_ref, b1_ref, w2_ref, yt_ref):
    rd = rd_ref[...]
    a = jnp.concatenate([a0_ref[...], a1_ref[...]], axis=1) * rd
    z = jnp.dot(a, w1_ref[...], preferred_element_type=jnp.float32)
    z = z + b1_ref[...]
    h = jnp.where(z > 0, z, jnp.exp(z) - 1.0)
    yt_ref[...] = jnp.dot(h * rd, w2_ref[...],
                          preferred_element_type=jnp.float32)


def _mid(a0, a1, rdeg, W1, b1, W2p):
    return pl.pallas_call(
        _mid_body,
        grid=(_GRID,),
        in_specs=[
            pl.BlockSpec((_R, DH), lambda i: (i, 0)),
            pl.BlockSpec((_R, DH), lambda i: (i, 0)),
            pl.BlockSpec((_R, 1), lambda i: (i, 0)),
            pl.BlockSpec((D, H), lambda i: (0, 0)),
            pl.BlockSpec((1, H), lambda i: (0, 0)),
            pl.BlockSpec((H, CP), lambda i: (0, 0)),
        ],
        out_specs=pl.BlockSpec((_R, CP), lambda i: (i, 0)),
        out_shape=jax.ShapeDtypeStruct((NPAD, CP), jnp.float32),
    )(a0, a1, rdeg, W1, b1, W2p)


def _scale_out_body(q0_ref, q1_ref, rd_ref, b2_ref, out_ref):
    out_ref[...] = (q0_ref[...] + q1_ref[...]) * rd_ref[...] + b2_ref[...]


def _scale_out(q0, q1, rdeg, b2p):
    return pl.pallas_call(
        _scale_out_body,
        grid=(_GRID,),
        in_specs=[
            pl.BlockSpec((_R, CP), lambda i: (i, 0)),
            pl.BlockSpec((_R, CP), lambda i: (i, 0)),
            pl.BlockSpec((_R, 1), lambda i: (i, 0)),
            pl.BlockSpec((1, CP), lambda i: (0, 0)),
        ],
        out_specs=pl.BlockSpec((_R, CP), lambda i: (i, 0)),
        out_shape=jax.ShapeDtypeStruct((NPAD, CP), jnp.float32),
    )(q0, q1, rdeg, b2p)


@jax.jit
def kernel(features, edge_index, W1, b1, W2, b2):
    epad = EPAD - E
    src_flat = jnp.pad(edge_index[0], (0, epad), constant_values=NPAD - 1)
    dst_flat = jnp.pad(edge_index[1], (0, epad), constant_values=NPAD - 1)
    src32 = src_flat.reshape(NTILE, KB, BB)
    dst32 = dst_flat.reshape(NTILE, KB, BB)
    src16 = src_flat.reshape(16, KBC, BB)
    dst16 = dst_flat.reshape(16, KBC, BB)
    x_pad = jnp.pad(features, ((0, NPAD - N), (0, 0)))
    W2p = jnp.pad(W2, ((0, 0), (0, CP - C)))
    b1r = b1.reshape(1, H)
    b2p = jnp.pad(b2, (0, CP - C)).reshape(1, CP)

    deg0, deg1 = _deg_kernel(dst32)
    xt0, xt1, rdeg = _scale_in(x_pad, deg0.reshape(NPAD, 1),
                               deg1.reshape(NPAD, 1))
    a0, a1 = _agg_d(src16, dst16, xt0, xt1)
    yt = _mid(a0, a1, rdeg, W1, b1r, W2p)
    q0, q1 = _agg_c(src32, dst32, yt)
    out = _scale_out(q0, q1, rdeg, b2p)
    return out[:N, :C]
